# baseline (speedup 1.0000x reference)
import functools

import jax
import jax.numpy as jnp
from jax import lax
from jax.experimental import pallas as pl
from jax.experimental.pallas import tpu as pltpu
from jax.experimental.pallas import tpu_sc as plsc

N_NODES = 10000
N_EDGES = 320000
HID = 128
HALF = 64
LAT = 64

NC = 2
NS = 16
CHUNK = 128
NBUF = 2
NCHUNK = 160
EPAD = NS * NCHUNK * CHUNK
NPAD = 10240
RPT = NPAD // NS
SR = 32
DEG_W = 16
RB = 512


def _sc_mesh():
    return plsc.VectorSubcoreMesh(core_axis_name="c", subcore_axis_name="s")


_SC_PARAMS = pltpu.CompilerParams(use_tc_tiling_on_sc=False)


def _sc_degree(dst3, zeros_deg, ones):
    half_chunks = NCHUNK // NC

    @functools.partial(
        pl.kernel,
        out_type=jax.ShapeDtypeStruct((NC, NPAD, DEG_W), jnp.float32),
        mesh=_sc_mesh(),
        compiler_params=_SC_PARAMS,
        scratch_types=[
            pltpu.VMEM((NCHUNK, CHUNK), jnp.int32),
            pltpu.VMEM((CHUNK, DEG_W), jnp.float32),
            pltpu.VMEM_SHARED((NPAD, DEG_W), jnp.float32),
            pltpu.SemaphoreType.DMA,
        ],
    )
    def deg_kernel(dst_hbm, z_hbm, ones_hbm, out_hbm, didx, ones_v, acc, sem):
        c = lax.axis_index("c")
        s = lax.axis_index("s")
        pltpu.sync_copy(z_hbm.at[pl.ds(s * RPT, RPT)], acc.at[pl.ds(s * RPT, RPT)])
        pltpu.sync_copy(dst_hbm.at[s], didx)
        pltpu.sync_copy(ones_hbm, ones_v)
        plsc.subcore_barrier()

        def body(j, carry):
            pltpu.async_copy(ones_v, acc.at[didx.at[c * half_chunks + j]],
                             sem, add=True)
            return carry

        lax.fori_loop(0, half_chunks, body, 0)

        def drain(j, carry):
            pltpu.make_async_copy(ones_v, acc.at[didx.at[0]], sem).wait()
            return carry

        lax.fori_loop(0, half_chunks, drain, 0)
        plsc.subcore_barrier()
        pltpu.sync_copy(acc.at[pl.ds(s * RPT, RPT)],
                        out_hbm.at[c, pl.ds(s * RPT, RPT)])

    return deg_kernel(dst3, zeros_deg, ones)


def _sc_fused(xs_split, src3, dst3, dinv16, b1c, zeros_half):
    @functools.partial(
        pl.kernel,
        out_type=(jax.ShapeDtypeStruct((NC, NPAD, HALF), jnp.float32),
                  jax.ShapeDtypeStruct((NC, NPAD, HALF), jnp.float32)),
        mesh=_sc_mesh(),
        compiler_params=_SC_PARAMS,
        scratch_types=[
            [pltpu.VMEM((NBUF, CHUNK), jnp.int32) for _ in range(2)],
            pltpu.VMEM((NCHUNK, CHUNK), jnp.int32),
            [pltpu.VMEM((CHUNK, HALF), jnp.float32) for _ in range(NBUF)],
            pltpu.VMEM((SR, HALF), jnp.float32),
            pltpu.VMEM((SR, HALF), jnp.float32),
            pltpu.VMEM((SR + 16,), jnp.float32),
            pltpu.VMEM((HALF // 16, 16), jnp.float32),
            pltpu.VMEM_SHARED((NPAD, HALF), jnp.float32),
            pltpu.VMEM_SHARED((NPAD, HALF), jnp.float32),
            pltpu.VMEM_SHARED((NPAD,), jnp.float32),
            [pltpu.SemaphoreType.DMA for _ in range(3)],
            [pltpu.SemaphoreType.DMA for _ in range(NBUF)],
            [pltpu.SemaphoreType.DMA for _ in range(NBUF)],
        ],
    )
    def fused_kernel(xs_hbm, src_hbm, dst_hbm, dinv_hbm, b1_hbm, z_hbm,
                     hs_hbm, raw2_hbm, sring, didx, rows,
                     buf_a, buf_x, buf_d, b1v, feat_s, acc, dinv_s,
                     isem, gsem, ssem):
        c = lax.axis_index("c")
        s = lax.axis_index("s")
        base = s * RPT
        init = [
            (z_hbm.at[pl.ds(base, RPT)], acc.at[pl.ds(base, RPT)]),
            (xs_hbm.at[c, pl.ds(base, RPT)], feat_s.at[pl.ds(base, RPT)]),
            (dst_hbm.at[s], didx),
            (dinv_hbm.at[pl.ds(base, RPT)], dinv_s.at[pl.ds(base, RPT)]),
            (b1_hbm.at[c], b1v),
        ]
        for a, b in init:
            pltpu.async_copy(a, b, isem[0])
        for a, b in init:
            pltpu.make_async_copy(a, b, isem[0]).wait()
        plsc.subcore_barrier()

        def prefetch(slot, g):
            off = jnp.minimum(g * NBUF, NCHUNK - NBUF)
            pltpu.async_copy(src_hbm.at[s, pl.ds(off, NBUF)],
                             sring[slot], isem[slot])

        def wait_prefetch(slot):
            pltpu.make_async_copy(src_hbm.at[s, pl.ds(0, NBUF)],
                                  sring[slot], isem[slot]).wait()

        def do_group(slot, g, first):
            wait_prefetch(slot)
            for b in range(NBUF):
                if not first:
                    pltpu.make_async_copy(rows[b], acc.at[didx.at[0]],
                                          ssem[b]).wait()
                pltpu.async_copy(feat_s.at[sring[slot].at[b]], rows[b], gsem[b])
            for b in range(NBUF):
                pltpu.make_async_copy(feat_s.at[sring[slot].at[b]], rows[b],
                                      gsem[b]).wait()
                pltpu.async_copy(rows[b], acc.at[didx.at[g * NBUF + b]],
                                 ssem[b], add=True)
            prefetch(slot, g + 2)

        def agg_pass():
            prefetch(0, 0)
            prefetch(1, 1)
            do_group(0, 0, True)
            do_group(1, 1, False)

            def body(i, carry):
                do_group(0, 2 * i, False)
                do_group(1, 2 * i + 1, False)
                return carry

            lax.fori_loop(1, NCHUNK // NBUF // 2, body, 0)
            for b in range(NBUF):
                pltpu.make_async_copy(rows[b], acc.at[didx.at[0]],
                                      ssem[b]).wait()
            for slot in range(2):
                wait_prefetch(slot)
            plsc.subcore_barrier()

        agg_pass()

        def strip(t, carry):
            r0 = base + t * SR
            pltpu.sync_copy(acc.at[pl.ds(r0, SR)], buf_a)
            pltpu.sync_copy(feat_s.at[pl.ds(r0, SR)], buf_x)
            pltpu.sync_copy(dinv_s.at[pl.ds(r0, SR)], buf_d.at[pl.ds(0, SR)])

            def row_body(r, rcarry):
                dvv = buf_d[pl.ds(r, 16)]
                dv = jnp.broadcast_to(dvv[0:1], (16,))
                for k in range(HALF // 16):
                    v = (buf_a[r, pl.ds(k * 16, 16)]
                         + buf_x[r, pl.ds(k * 16, 16)]) * dv + b1v[k]
                    buf_x[r, pl.ds(k * 16, 16)] = jnp.maximum(v, 0.0) * dv
                return rcarry

            lax.fori_loop(0, SR, row_body, 0)
            pltpu.sync_copy(buf_x, feat_s.at[pl.ds(r0, SR)])
            return carry

        lax.fori_loop(0, RPT // SR, strip, 0)
        pltpu.async_copy(feat_s.at[pl.ds(base, RPT)],
                         hs_hbm.at[c, pl.ds(base, RPT)], isem[2])
        pltpu.sync_copy(z_hbm.at[pl.ds(base, RPT)], acc.at[pl.ds(base, RPT)])
        plsc.subcore_barrier()

        agg_pass()
        pltpu.sync_copy(acc.at[pl.ds(base, RPT)],
                        raw2_hbm.at[c, pl.ds(base, RPT)])
        pltpu.make_async_copy(feat_s.at[pl.ds(base, RPT)],
                              hs_hbm.at[c, pl.ds(base, RPT)], isem[2]).wait()

    return fused_kernel(xs_split, src3, dst3, dinv16, b1c, zeros_half)


def _dinv_block(degp_blk):
    deg = degp_blk[0, :, 0:1] + degp_blk[1, :, 0:1] + 1.0
    return lax.rsqrt(deg)


def _tc_xs(xp, W1, degp):
    def body(x_ref, w_ref, dp_ref, o_ref, d_ref):
        dinv = _dinv_block(dp_ref[...])
        xw = jnp.dot(x_ref[...], w_ref[...],
                     preferred_element_type=jnp.float32) * dinv
        o_ref[0] = xw[:, :HALF]
        o_ref[1] = xw[:, HALF:]
        d_ref[...] = jnp.broadcast_to(dinv, (RB, DEG_W))

    return pl.pallas_call(
        body,
        grid=(NPAD // RB,),
        in_specs=[
            pl.BlockSpec((RB, HID), lambda i: (i, 0)),
            pl.BlockSpec((HID, HID), lambda i: (0, 0)),
            pl.BlockSpec((NC, RB, DEG_W), lambda i: (0, i, 0)),
        ],
        out_specs=[
            pl.BlockSpec((NC, RB, HALF), lambda i: (0, i, 0)),
            pl.BlockSpec((RB, DEG_W), lambda i: (i, 0)),
        ],
        out_shape=[
            jax.ShapeDtypeStruct((NC, NPAD, HALF), jnp.float32),
            jax.ShapeDtypeStruct((NPAD, DEG_W), jnp.float32),
        ],
    )(xp, W1, degp)


def _tc_out(raw, hs_split, dinv16, Wcat, bcat):
    def body(r_ref, hs_ref, dv_ref, w_ref, b_ref, o_ref):
        dinv = dv_ref[:, 0:1]
        raw_full = jnp.concatenate([r_ref[0], r_ref[1]], axis=1)
        hs = jnp.concatenate([hs_ref[0], hs_ref[1]], axis=1)
        z = (raw_full + hs) * dinv
        o_ref[...] = jnp.dot(z, w_ref[...],
                             preferred_element_type=jnp.float32) + b_ref[...]

    return pl.pallas_call(
        body,
        grid=(NPAD // RB,),
        in_specs=[
            pl.BlockSpec((NC, RB, HALF), lambda i: (0, i, 0)),
            pl.BlockSpec((NC, RB, HALF), lambda i: (0, i, 0)),
            pl.BlockSpec((RB, DEG_W), lambda i: (i, 0)),
            pl.BlockSpec((HID, 2 * LAT), lambda i: (0, 0)),
            pl.BlockSpec((1, 2 * LAT), lambda i: (0, 0)),
        ],
        out_specs=pl.BlockSpec((RB, 2 * LAT), lambda i: (i, 0)),
        out_shape=jax.ShapeDtypeStruct((NPAD, 2 * LAT), jnp.float32),
    )(raw, hs_split, dinv16, Wcat, bcat)


def kernel(x, edge_index, W1, b1, Wmu, bmu, Wls, bls):
    f32 = jnp.float32
    e32 = jnp.concatenate(
        [edge_index.astype(jnp.int32),
         jnp.full((2, EPAD - N_EDGES), N_NODES, jnp.int32)], axis=1)
    dst3 = e32[1].reshape(NS, NCHUNK, CHUNK)
    src3 = e32[0].reshape(NS, NCHUNK, CHUNK)

    xp = jnp.concatenate([x.astype(f32),
                          jnp.zeros((NPAD - N_NODES, HID), f32)], axis=0)
    zeros_deg = jnp.zeros((NPAD, DEG_W), f32)
    zeros_half = jnp.zeros((NPAD, HALF), f32)
    ones = jnp.ones((CHUNK, DEG_W), f32)

    degp = _sc_degree(dst3, zeros_deg, ones)
    xs_split, dinv16 = _tc_xs(xp, W1, degp)
    b1c = b1.reshape(NC, HALF // 16, 16)
    hs_split, raw2 = _sc_fused(xs_split, src3, dst3, dinv16[:, 0], b1c,
                               zeros_half)
    Wcat = jnp.concatenate([Wmu, Wls], axis=1)
    bcat = jnp.concatenate([bmu, bls]).reshape(1, 2 * LAT)
    zc = _tc_out(raw2, hs_split, dinv16, Wcat, bcat)
    return zc[:N_NODES, :LAT], zc[:N_NODES, LAT:]


# speedup vs baseline: 1.0048x; 1.0048x over previous
import functools

import jax
import jax.numpy as jnp
from jax import lax
from jax.experimental import pallas as pl
from jax.experimental.pallas import tpu as pltpu
from jax.experimental.pallas import tpu_sc as plsc

N_NODES = 10000
N_EDGES = 320000
HID = 128
HALF = 64
LAT = 64

NC = 2
NS = 16
CHUNK = 128
NBUF = 2
NCHUNK = 160
EPAD = NS * NCHUNK * CHUNK
NPAD = 10240
RPT = NPAD // NS
SR = 64
DEG_W = 16
RB = 512


def _sc_mesh():
    return plsc.VectorSubcoreMesh(core_axis_name="c", subcore_axis_name="s")


_SC_PARAMS = pltpu.CompilerParams(use_tc_tiling_on_sc=False)


def _sc_degree(dst3, zeros_deg, ones):
    half_chunks = NCHUNK // NC

    @functools.partial(
        pl.kernel,
        out_type=jax.ShapeDtypeStruct((NC, NPAD, DEG_W), jnp.float32),
        mesh=_sc_mesh(),
        compiler_params=_SC_PARAMS,
        scratch_types=[
            pltpu.VMEM((NCHUNK, CHUNK), jnp.int32),
            pltpu.VMEM((CHUNK, DEG_W), jnp.float32),
            pltpu.VMEM_SHARED((NPAD, DEG_W), jnp.float32),
            pltpu.SemaphoreType.DMA,
        ],
    )
    def deg_kernel(dst_hbm, z_hbm, ones_hbm, out_hbm, didx, ones_v, acc, sem):
        c = lax.axis_index("c")
        s = lax.axis_index("s")
        pltpu.sync_copy(z_hbm.at[pl.ds(s * RPT, RPT)], acc.at[pl.ds(s * RPT, RPT)])
        pltpu.sync_copy(dst_hbm.at[s], didx)
        pltpu.sync_copy(ones_hbm, ones_v)
        plsc.subcore_barrier()

        def body(j, carry):
            pltpu.async_copy(ones_v, acc.at[didx.at[c * half_chunks + j]],
                             sem, add=True)
            return carry

        lax.fori_loop(0, half_chunks, body, 0)

        def drain(j, carry):
            pltpu.make_async_copy(ones_v, acc.at[didx.at[0]], sem).wait()
            return carry

        lax.fori_loop(0, half_chunks, drain, 0)
        plsc.subcore_barrier()
        pltpu.sync_copy(acc.at[pl.ds(s * RPT, RPT)],
                        out_hbm.at[c, pl.ds(s * RPT, RPT)])

    return deg_kernel(dst3, zeros_deg, ones)


def _sc_fused(xs_split, src3, dst3, dinv16, b1c, zeros_half):
    @functools.partial(
        pl.kernel,
        out_type=(jax.ShapeDtypeStruct((NC, NPAD, HALF), jnp.float32),
                  jax.ShapeDtypeStruct((NC, NPAD, HALF), jnp.float32)),
        mesh=_sc_mesh(),
        compiler_params=_SC_PARAMS,
        scratch_types=[
            [pltpu.VMEM((NBUF, CHUNK), jnp.int32) for _ in range(2)],
            pltpu.VMEM((NCHUNK, CHUNK), jnp.int32),
            [pltpu.VMEM((CHUNK, HALF), jnp.float32) for _ in range(NBUF)],
            pltpu.VMEM((SR, HALF), jnp.float32),
            pltpu.VMEM((SR, HALF), jnp.float32),
            pltpu.VMEM((SR + 16,), jnp.float32),
            pltpu.VMEM((HALF // 16, 16), jnp.float32),
            pltpu.VMEM_SHARED((NPAD, HALF), jnp.float32),
            pltpu.VMEM_SHARED((NPAD, HALF), jnp.float32),
            pltpu.VMEM_SHARED((NPAD,), jnp.float32),
            [pltpu.SemaphoreType.DMA for _ in range(3)],
            [pltpu.SemaphoreType.DMA for _ in range(NBUF)],
            [pltpu.SemaphoreType.DMA for _ in range(NBUF)],
        ],
    )
    def fused_kernel(xs_hbm, src_hbm, dst_hbm, dinv_hbm, b1_hbm, z_hbm,
                     hs_hbm, raw2_hbm, sring, didx, rows,
                     buf_a, buf_x, buf_d, b1v, feat_s, acc, dinv_s,
                     isem, gsem, ssem):
        c = lax.axis_index("c")
        s = lax.axis_index("s")
        base = s * RPT
        init = [
            (z_hbm.at[pl.ds(base, RPT)], acc.at[pl.ds(base, RPT)]),
            (xs_hbm.at[c, pl.ds(base, RPT)], feat_s.at[pl.ds(base, RPT)]),
            (dst_hbm.at[s], didx),
            (dinv_hbm.at[pl.ds(base, RPT)], dinv_s.at[pl.ds(base, RPT)]),
            (b1_hbm.at[c], b1v),
        ]
        for a, b in init:
            pltpu.async_copy(a, b, isem[0])
        for a, b in init:
            pltpu.make_async_copy(a, b, isem[0]).wait()
        plsc.subcore_barrier()

        def prefetch(slot, g):
            off = jnp.minimum(g * NBUF, NCHUNK - NBUF)
            pltpu.async_copy(src_hbm.at[s, pl.ds(off, NBUF)],
                             sring[slot], isem[slot])

        def wait_prefetch(slot):
            pltpu.make_async_copy(src_hbm.at[s, pl.ds(0, NBUF)],
                                  sring[slot], isem[slot]).wait()

        def do_group(slot, g, first):
            wait_prefetch(slot)
            for b in range(NBUF):
                if not first:
                    pltpu.make_async_copy(rows[b], acc.at[didx.at[0]],
                                          ssem[b]).wait()
                pltpu.async_copy(feat_s.at[sring[slot].at[b]], rows[b], gsem[b])
            for b in range(NBUF):
                pltpu.make_async_copy(feat_s.at[sring[slot].at[b]], rows[b],
                                      gsem[b]).wait()
                pltpu.async_copy(rows[b], acc.at[didx.at[g * NBUF + b]],
                                 ssem[b], add=True)
            prefetch(slot, g + 2)

        def agg_pass():
            prefetch(0, 0)
            prefetch(1, 1)
            do_group(0, 0, True)
            do_group(1, 1, False)

            def body(i, carry):
                do_group(0, 2 * i, False)
                do_group(1, 2 * i + 1, False)
                return carry

            lax.fori_loop(1, NCHUNK // NBUF // 2, body, 0)
            for b in range(NBUF):
                pltpu.make_async_copy(rows[b], acc.at[didx.at[0]],
                                      ssem[b]).wait()
            for slot in range(2):
                wait_prefetch(slot)
            plsc.subcore_barrier()

        agg_pass()

        def strip(t, carry):
            r0 = base + t * SR
            pltpu.sync_copy(acc.at[pl.ds(r0, SR)], buf_a)
            pltpu.sync_copy(feat_s.at[pl.ds(r0, SR)], buf_x)
            pltpu.sync_copy(dinv_s.at[pl.ds(r0, SR)], buf_d.at[pl.ds(0, SR)])

            def row_body(r, rcarry):
                dvv = buf_d[pl.ds(r, 16)]
                dv = jnp.broadcast_to(dvv[0:1], (16,))
                for k in range(HALF // 16):
                    v = (buf_a[r, pl.ds(k * 16, 16)]
                         + buf_x[r, pl.ds(k * 16, 16)]) * dv + b1v[k]
                    buf_x[r, pl.ds(k * 16, 16)] = jnp.maximum(v, 0.0) * dv
                return rcarry

            lax.fori_loop(0, SR, row_body, 0)
            pltpu.sync_copy(buf_x, feat_s.at[pl.ds(r0, SR)])
            return carry

        lax.fori_loop(0, RPT // SR, strip, 0)
        pltpu.async_copy(feat_s.at[pl.ds(base, RPT)],
                         hs_hbm.at[c, pl.ds(base, RPT)], isem[2])
        pltpu.sync_copy(z_hbm.at[pl.ds(base, RPT)], acc.at[pl.ds(base, RPT)])
        plsc.subcore_barrier()

        agg_pass()
        pltpu.sync_copy(acc.at[pl.ds(base, RPT)],
                        raw2_hbm.at[c, pl.ds(base, RPT)])
        pltpu.make_async_copy(feat_s.at[pl.ds(base, RPT)],
                              hs_hbm.at[c, pl.ds(base, RPT)], isem[2]).wait()

    return fused_kernel(xs_split, src3, dst3, dinv16, b1c, zeros_half)


def _dinv_block(degp_blk):
    deg = degp_blk[0, :, 0:1] + degp_blk[1, :, 0:1] + 1.0
    return lax.rsqrt(deg)


def _tc_xs(xp, W1, degp):
    def body(x_ref, w_ref, dp_ref, o_ref, d_ref):
        dinv = _dinv_block(dp_ref[...])
        xw = jnp.dot(x_ref[...], w_ref[...],
                     preferred_element_type=jnp.float32) * dinv
        o_ref[0] = xw[:, :HALF]
        o_ref[1] = xw[:, HALF:]
        d_ref[...] = jnp.broadcast_to(dinv, (RB, DEG_W))

    return pl.pallas_call(
        body,
        grid=(NPAD // RB,),
        in_specs=[
            pl.BlockSpec((RB, HID), lambda i: (i, 0)),
            pl.BlockSpec((HID, HID), lambda i: (0, 0)),
            pl.BlockSpec((NC, RB, DEG_W), lambda i: (0, i, 0)),
        ],
        out_specs=[
            pl.BlockSpec((NC, RB, HALF), lambda i: (0, i, 0)),
            pl.BlockSpec((RB, DEG_W), lambda i: (i, 0)),
        ],
        out_shape=[
            jax.ShapeDtypeStruct((NC, NPAD, HALF), jnp.float32),
            jax.ShapeDtypeStruct((NPAD, DEG_W), jnp.float32),
        ],
    )(xp, W1, degp)


def _tc_out(raw, hs_split, dinv16, Wcat, bcat):
    def body(r_ref, hs_ref, dv_ref, w_ref, b_ref, o_ref):
        dinv = dv_ref[:, 0:1]
        raw_full = jnp.concatenate([r_ref[0], r_ref[1]], axis=1)
        hs = jnp.concatenate([hs_ref[0], hs_ref[1]], axis=1)
        z = (raw_full + hs) * dinv
        o_ref[...] = jnp.dot(z, w_ref[...],
                             preferred_element_type=jnp.float32) + b_ref[...]

    return pl.pallas_call(
        body,
        grid=(NPAD // RB,),
        in_specs=[
            pl.BlockSpec((NC, RB, HALF), lambda i: (0, i, 0)),
            pl.BlockSpec((NC, RB, HALF), lambda i: (0, i, 0)),
            pl.BlockSpec((RB, DEG_W), lambda i: (i, 0)),
            pl.BlockSpec((HID, 2 * LAT), lambda i: (0, 0)),
            pl.BlockSpec((1, 2 * LAT), lambda i: (0, 0)),
        ],
        out_specs=pl.BlockSpec((RB, 2 * LAT), lambda i: (i, 0)),
        out_shape=jax.ShapeDtypeStruct((NPAD, 2 * LAT), jnp.float32),
    )(raw, hs_split, dinv16, Wcat, bcat)


def kernel(x, edge_index, W1, b1, Wmu, bmu, Wls, bls):
    f32 = jnp.float32
    e32 = jnp.concatenate(
        [edge_index.astype(jnp.int32),
         jnp.full((2, EPAD - N_EDGES), N_NODES, jnp.int32)], axis=1)
    dst3 = e32[1].reshape(NS, NCHUNK, CHUNK)
    src3 = e32[0].reshape(NS, NCHUNK, CHUNK)

    xp = jnp.concatenate([x.astype(f32),
                          jnp.zeros((NPAD - N_NODES, HID), f32)], axis=0)
    zeros_deg = jnp.zeros((NPAD, DEG_W), f32)
    zeros_half = jnp.zeros((NPAD, HALF), f32)
    ones = jnp.ones((CHUNK, DEG_W), f32)

    degp = _sc_degree(dst3, zeros_deg, ones)
    xs_split, dinv16 = _tc_xs(xp, W1, degp)
    b1c = b1.reshape(NC, HALF // 16, 16)
    hs_split, raw2 = _sc_fused(xs_split, src3, dst3, dinv16[:, 0], b1c,
                               zeros_half)
    Wcat = jnp.concatenate([Wmu, Wls], axis=1)
    bcat = jnp.concatenate([bmu, bls]).reshape(1, 2 * LAT)
    zc = _tc_out(raw2, hs_split, dinv16, Wcat, bcat)
    return zc[:N_NODES, :LAT], zc[:N_NODES, LAT:]
